# baseline (speedup 1.0000x reference)
import functools

import jax
import jax.numpy as jnp
from jax import lax
from jax.experimental import pallas as pl
from jax.experimental.pallas import tpu as pltpu
from jax.experimental.pallas import tpu_sc as plsc

N = 10000
E = 160000
H = 256
G = 16
C = 5
F_IN = 5

N_PAD = 10240
ROWS_PER_TILE = N_PAD // 16
E_PAD = 163840
CHUNK = 128
J_EDGE = 40
CHUNK_C = 112
J_COL = 128
GRP_COL = 8
CHUNK_COL = 80
E_PAD_C = 161280
BLK = 512
N_BLKS = N_PAD // BLK

_MESH = dict(core_axis_name="c", subcore_axis_name="s", num_cores=2,
             num_subcores=16)
_PREC = lax.Precision.DEFAULT


def _fill_rows(ref, nrows, width, value):
    per_row = width // 16

    def body(i, _):
        r = i // per_row
        k = i % per_row
        ref[r, pl.ds(k * 16, 16)] = jnp.full((16,), value, jnp.float32)
        return 0

    lax.fori_loop(0, nrows * per_row, body, 0)


def _deg_body(dst_hbm, degp_hbm, acc, idx_v, ones_v, zbuf, buf):
    c = lax.axis_index("c")
    s = lax.axis_index("s")
    w = c * 16 + s
    _fill_rows(ones_v, CHUNK, 16, 1.0)
    _fill_rows(zbuf, CHUNK, 16, 0.0)
    base = s * ROWS_PER_TILE
    for k in range(ROWS_PER_TILE // CHUNK):
        pltpu.sync_copy(zbuf, acc.at[pl.ds(base + k * CHUNK, CHUNK)])
    plsc.subcore_barrier()
    pltpu.sync_copy(dst_hbm.at[w], idx_v)

    def body(j, _):
        pltpu.sync_copy(ones_v, acc.at[idx_v.at[j]], add=True)
        return 0

    lax.fori_loop(0, J_EDGE, body, 0)
    plsc.subcore_barrier()
    for k in range(ROWS_PER_TILE // CHUNK):
        sl = pl.ds(base + k * CHUNK, CHUNK)
        pltpu.sync_copy(acc.at[sl], buf)
        pltpu.sync_copy(buf, degp_hbm.at[c].at[sl])


def _deg(dst_blocks):
    f = pl.kernel(
        _deg_body,
        out_type=jax.ShapeDtypeStruct((2, N_PAD, 16), jnp.float32),
        mesh=plsc.VectorSubcoreMesh(**_MESH),
        compiler_params=pltpu.CompilerParams(use_tc_tiling_on_sc=False),
        scratch_types=[
            pltpu.VMEM_SHARED((N_PAD, 16), jnp.float32),
            pltpu.VMEM((J_EDGE, CHUNK), jnp.int32),
            pltpu.VMEM((CHUNK, 16), jnp.float32),
            pltpu.VMEM((CHUNK, 16), jnp.float32),
            pltpu.VMEM((CHUNK, 16), jnp.float32),
        ],
    )
    return f(dst_blocks)


J_EDGE_C = 45
GRP_E = 9


def _make_mp_async_body(width, jt, grp, edge_split, nbuf, chunk):
    ngrp = jt // grp
    assert jt % grp == 0 and grp % nbuf == 0

    def body(g_hbm, src_hbm, dst_hbm, out_hbm, acc, src_v, dst_v,
             rows, gsems, ssems):
        c = lax.axis_index("c")
        s = lax.axis_index("s")
        tile = c * 16 + s if edge_split else s
        g_c = g_hbm if edge_split else g_hbm.at[c]
        base = s * ROWS_PER_TILE
        sl_all = pl.ds(base, ROWS_PER_TILE)

        if edge_split:
            @pl.when(c == 0)
            def _():
                pltpu.sync_copy(g_c.at[sl_all], acc.at[sl_all])

            @pl.when(c == 1)
            def _():
                _fill_rows(rows[0], chunk, width, 0.0)
                nfull = ROWS_PER_TILE // chunk
                rem = ROWS_PER_TILE - nfull * chunk
                for k in range(nfull):
                    pltpu.sync_copy(
                        rows[0], acc.at[pl.ds(base + k * chunk, chunk)])
                if rem:
                    pltpu.sync_copy(
                        rows[0].at[pl.ds(0, rem)],
                        acc.at[pl.ds(base + nfull * chunk, rem)])
        else:
            pltpu.sync_copy(g_c.at[sl_all], acc.at[sl_all])
        plsc.subcore_barrier()

        def load_grp(grp_i, slot):
            gsl = pl.ds(grp_i * grp, grp)
            osl = pl.ds(slot * grp, grp)
            pltpu.sync_copy(src_hbm.at[tile].at[gsl], src_v.at[osl])
            pltpu.sync_copy(dst_hbm.at[tile].at[gsl], dst_v.at[osl])

        def start_gather(row, b):
            pltpu.async_copy(g_c.at[src_v.at[row]], rows[b], gsems[b])

        def wait_gather(row, b):
            pltpu.make_async_copy(g_c.at[src_v.at[row]], rows[b],
                                  gsems[b]).wait()

        def start_scatter(row, b):
            pltpu.async_copy(rows[b], acc.at[dst_v.at[row]], ssems[b],
                             add=True)

        def wait_scatter(row, b):
            pltpu.make_async_copy(rows[b], acc.at[dst_v.at[row]],
                                  ssems[b]).wait()

        load_grp(0, 0)
        start_gather(0, 0)
        start_gather(1, 1)

        def grp_body(g, _):
            slot = g % 2
            nslot = (g + 1) % 2

            @pl.when(g >= 1)
            def _():
                for d in range(nbuf - 2):
                    wait_scatter(slot * grp, (grp - (nbuf - 2) + d) % nbuf)

            @pl.when(g <= ngrp - 2)
            def _():
                load_grp(g + 1, nslot)

            for j in range(grp):
                b = j % nbuf
                row = slot * grp + j
                wait_gather(row, b)
                start_scatter(row, b)
                nb = (j + 2) % nbuf
                if j + 2 < grp:
                    if j >= nbuf - 2:
                        wait_scatter(row, nb)
                    start_gather(slot * grp + j + 2, nb)
                else:
                    @pl.when(g <= ngrp - 2)
                    def _(j=j, nb=nb, row=row):
                        wait_scatter(row, nb)
                        start_gather(nslot * grp + (j + 2 - grp), nb)
            return 0

        lax.fori_loop(0, ngrp, grp_body, 0)
        last = ((ngrp - 1) % 2) * grp
        for j in range(grp - nbuf, grp):
            wait_scatter(last + j, j % nbuf)

        plsc.subcore_barrier()
        pltpu.sync_copy(acc.at[sl_all], out_hbm.at[c].at[sl_all])

    return body


def _mp_call(width, jt, grp, edge_split, nbuf, chunk):
    return pl.kernel(
        _make_mp_async_body(width, jt, grp, edge_split, nbuf, chunk),
        out_type=jax.ShapeDtypeStruct((2, N_PAD, width), jnp.float32),
        mesh=plsc.VectorSubcoreMesh(**_MESH),
        compiler_params=pltpu.CompilerParams(use_tc_tiling_on_sc=False),
        scratch_types=[
            pltpu.VMEM_SHARED((N_PAD, width), jnp.float32),
            pltpu.VMEM((2 * grp, chunk), jnp.int32),
            pltpu.VMEM((2 * grp, chunk), jnp.int32),
            [pltpu.VMEM((chunk, width), jnp.float32) for _ in range(nbuf)],
            [pltpu.SemaphoreType.DMA for _ in range(nbuf)],
            [pltpu.SemaphoreType.DMA for _ in range(nbuf)],
        ],
    )


def _mp_edge(g, src_blocks, dst_blocks):
    return _mp_call(16, J_EDGE_C, GRP_E, True, 3, CHUNK_C)(
        g, src_blocks, dst_blocks)


def _mp_col(g, src_blocks, dst_blocks):
    return _mp_call(128, J_COL, GRP_COL, False, 4, CHUNK_COL)(
        g, src_blocks, dst_blocks)


def _pre_body(degp_ref, x_ref, dinv_ref, g0_ref):
    p = degp_ref[...]
    deg = 1.0 + p[0, :, 0:1] + p[1, :, 0:1]
    dinv = lax.rsqrt(deg)
    dinv_ref[...] = dinv
    g0_ref[...] = x_ref[...] * dinv


def _pre(degp, x_pad):
    return pl.pallas_call(
        _pre_body,
        grid=(N_BLKS,),
        in_specs=[
            pl.BlockSpec((2, BLK, 16), lambda i: (0, i, 0)),
            pl.BlockSpec((BLK, 16), lambda i: (i, 0)),
        ],
        out_specs=[
            pl.BlockSpec((BLK, 1), lambda i: (i, 0)),
            pl.BlockSpec((BLK, 16), lambda i: (i, 0)),
        ],
        out_shape=[
            jax.ShapeDtypeStruct((N_PAD, 1), jnp.float32),
            jax.ShapeDtypeStruct((N_PAD, 16), jnp.float32),
        ],
    )(degp, x_pad)


def _make_layer_body(first):
    def body(t_ref, dinv_ref, w_ref, b_ref, g_ref):
        p = t_ref[...]
        t = (p[0] + p[1]) if first else jnp.concatenate([p[0], p[1]], axis=1)
        dinv = dinv_ref[...]
        a = t * dinv
        h = jnp.dot(a, w_ref[...], preferred_element_type=jnp.float32,
                    precision=_PREC) + b_ref[...]
        g = jnp.maximum(h, 0.0) * dinv
        g_ref[0, :, :] = g[:, :128]
        g_ref[1, :, :] = g[:, 128:]

    return body


def _layer(t, dinv, w, b, first):
    kin = w.shape[0]
    return pl.pallas_call(
        _make_layer_body(first),
        grid=(N_BLKS,),
        in_specs=[
            pl.BlockSpec((2, BLK, kin if first else 128),
                         lambda i: (0, i, 0)),
            pl.BlockSpec((BLK, 1), lambda i: (i, 0)),
            pl.BlockSpec(w.shape, lambda i: (0, 0)),
            pl.BlockSpec((1, H), lambda i: (0, 0)),
        ],
        out_specs=pl.BlockSpec((2, BLK, 128), lambda i: (0, i, 0)),
        out_shape=jax.ShapeDtypeStruct((2, N_PAD, 128), jnp.float32),
    )(t, dinv, w, b)


def _final_body(t_ref, dinv_ref, w_ref, b_ref, batch_ref, wc1_ref, bc1_ref,
                wc2_ref, bc2_ref, out_ref, sum_s, max_s, cnt_s):
    i = pl.program_id(0)

    @pl.when(i == 0)
    def _():
        sum_s[...] = jnp.zeros_like(sum_s)
        cnt_s[...] = jnp.zeros_like(cnt_s)
        max_s[...] = jnp.full_like(max_s, -jnp.inf)

    p = t_ref[...]
    t = jnp.concatenate([p[0], p[1]], axis=1)
    h = jnp.dot(t * dinv_ref[...], w_ref[...],
                preferred_element_type=jnp.float32, precision=_PREC)
    h = jnp.maximum(h + b_ref[...], 0.0)
    b = batch_ref[...]
    onehot = (b == lax.broadcasted_iota(jnp.int32, (1, G), 1)).astype(
        jnp.float32)
    sum_s[...] += lax.dot_general(onehot, h, (((0,), (0,)), ((), ())),
                                  preferred_element_type=jnp.float32,
                                  precision=_PREC)
    cnt_s[...] += lax.dot_general(onehot, jnp.ones((BLK, 1), jnp.float32),
                                  (((0,), (0,)), ((), ())),
                                  preferred_element_type=jnp.float32,
                                  precision=_PREC)
    def upd_max(g, _):
        col = jnp.max(jnp.where(b == g, h, -jnp.inf), axis=0, keepdims=True)
        max_s[pl.ds(g, 1), :] = jnp.maximum(max_s[pl.ds(g, 1), :], col)
        return 0

    lax.fori_loop(b[0, 0], jnp.minimum(b[BLK - 1, 0], G - 1) + 1, upd_max, 0)

    @pl.when(i == N_BLKS - 1)
    def _():
        mean = sum_s[...] / jnp.maximum(cnt_s[...], 1.0)
        mx = max_s[...]
        mx = jnp.where(jnp.isfinite(mx), mx, 0.0)
        pooled = jnp.concatenate([mean, mx], axis=1)
        hid = jnp.dot(pooled, wc1_ref[...], preferred_element_type=jnp.float32,
                      precision=_PREC) + bc1_ref[...]
        hid = jnp.maximum(hid, 0.0)
        out_ref[...] = jnp.dot(hid, wc2_ref[...],
                               preferred_element_type=jnp.float32,
                               precision=_PREC) + bc2_ref[...]


def _final(t3, dinv, w3, b3, batch_pad, wc1, bc1, wc2p, bc2p):
    return pl.pallas_call(
        _final_body,
        grid=(N_BLKS,),
        in_specs=[
            pl.BlockSpec((2, BLK, 128), lambda i: (0, i, 0)),
            pl.BlockSpec((BLK, 1), lambda i: (i, 0)),
            pl.BlockSpec((H, H), lambda i: (0, 0)),
            pl.BlockSpec((1, H), lambda i: (0, 0)),
            pl.BlockSpec((BLK, 1), lambda i: (i, 0)),
            pl.BlockSpec((2 * H, H), lambda i: (0, 0)),
            pl.BlockSpec((1, H), lambda i: (0, 0)),
            pl.BlockSpec((H, 128), lambda i: (0, 0)),
            pl.BlockSpec((1, 128), lambda i: (0, 0)),
        ],
        out_specs=pl.BlockSpec((G, 128), lambda i: (0, 0)),
        out_shape=jax.ShapeDtypeStruct((G, 128), jnp.float32),
        scratch_shapes=[
            pltpu.VMEM((G, H), jnp.float32),
            pltpu.VMEM((G, H), jnp.float32),
            pltpu.VMEM((G, 1), jnp.float32),
        ],
    )(t3, dinv, w3, b3, batch_pad, wc1, bc1, wc2p, bc2p)


@jax.jit
def kernel(x, edge_index, batch, W1, b1, W2, b2, W3, b3, Wc1, bc1, Wc2, bc2):
    src = edge_index[0]
    dst = edge_index[1]
    pad = jnp.full((E_PAD - E,), N, jnp.int32)
    srcp = jnp.concatenate([src, pad])
    dstp = jnp.concatenate([dst, pad])
    dst_deg = dstp.reshape(32, J_EDGE, CHUNK)
    src_c = srcp.reshape(16, J_COL, CHUNK_COL)
    dst_c = dstp.reshape(16, J_COL, CHUNK_COL)
    pad_c = jnp.full((E_PAD_C - E,), N, jnp.int32)
    src_e = jnp.concatenate([src, pad_c]).reshape(32, J_EDGE_C, CHUNK_C)
    dst_e = jnp.concatenate([dst, pad_c]).reshape(32, J_EDGE_C, CHUNK_C)

    x_pad = jnp.zeros((N_PAD, 16), jnp.float32).at[:N, :F_IN].set(x)
    batch_pad = jnp.full((N_PAD, 1), G, jnp.int32).at[:N, 0].set(batch)
    W1p = jnp.zeros((16, H), jnp.float32).at[:F_IN].set(W1)
    Wc2p = jnp.zeros((H, 128), jnp.float32).at[:, :C].set(Wc2)
    bc2p = jnp.zeros((1, 128), jnp.float32).at[0, :C].set(bc2)

    degp = _deg(dst_deg)
    dinv, g0 = _pre(degp, x_pad)
    t1 = _mp_edge(g0, src_e, dst_e)
    g1 = _layer(t1, dinv, W1p, b1.reshape(1, H), first=True)
    t2 = _mp_col(g1, src_c, dst_c)
    g2 = _layer(t2, dinv, W2, b2.reshape(1, H), first=False)
    t3 = _mp_col(g2, src_c, dst_c)
    out = _final(t3, dinv, W3, b3.reshape(1, H), batch_pad,
                 Wc1, bc1.reshape(1, H), Wc2p, bc2p)
    return out[:, :C]


# speedup vs baseline: 1.5494x; 1.5494x over previous
import functools

import jax
import jax.numpy as jnp
from jax import lax
from jax.experimental import pallas as pl
from jax.experimental.pallas import tpu as pltpu
from jax.experimental.pallas import tpu_sc as plsc

N = 10000
E = 160000
H = 256
G = 16
C = 5
F_IN = 5

N_PAD = 10240
ROWS_PER_TILE = N_PAD // 16
E_PAD = 163840
CHUNK = 128
J_EDGE = 40
CHUNK_C = 112
J_COL = 90
GRP_COL = 6
CHUNK_COL = 112
E_PAD_C = 161280
BLK = 512
N_BLKS = N_PAD // BLK

_MESH = dict(core_axis_name="c", subcore_axis_name="s", num_cores=2,
             num_subcores=16)
_PREC = lax.Precision.DEFAULT


def _fill_rows(ref, nrows, width, value):
    per_row = width // 16

    def body(i, _):
        r = i // per_row
        k = i % per_row
        ref[r, pl.ds(k * 16, 16)] = jnp.full((16,), value, jnp.float32)
        return 0

    lax.fori_loop(0, nrows * per_row, body, 0)


def _deg_body(dst_hbm, degp_hbm, acc, idx_v, ones_v, zbuf, buf):
    c = lax.axis_index("c")
    s = lax.axis_index("s")
    w = c * 16 + s
    _fill_rows(ones_v, CHUNK, 16, 1.0)
    _fill_rows(zbuf, CHUNK, 16, 0.0)
    base = s * ROWS_PER_TILE
    for k in range(ROWS_PER_TILE // CHUNK):
        pltpu.sync_copy(zbuf, acc.at[pl.ds(base + k * CHUNK, CHUNK)])
    plsc.subcore_barrier()
    pltpu.sync_copy(dst_hbm.at[w], idx_v)

    def body(j, _):
        pltpu.sync_copy(ones_v, acc.at[idx_v.at[j]], add=True)
        return 0

    lax.fori_loop(0, J_EDGE, body, 0)
    plsc.subcore_barrier()
    for k in range(ROWS_PER_TILE // CHUNK):
        sl = pl.ds(base + k * CHUNK, CHUNK)
        pltpu.sync_copy(acc.at[sl], buf)
        pltpu.sync_copy(buf, degp_hbm.at[c].at[sl])


def _deg(dst_blocks):
    f = pl.kernel(
        _deg_body,
        out_type=jax.ShapeDtypeStruct((2, N_PAD, 16), jnp.float32),
        mesh=plsc.VectorSubcoreMesh(**_MESH),
        compiler_params=pltpu.CompilerParams(use_tc_tiling_on_sc=False),
        scratch_types=[
            pltpu.VMEM_SHARED((N_PAD, 16), jnp.float32),
            pltpu.VMEM((J_EDGE, CHUNK), jnp.int32),
            pltpu.VMEM((CHUNK, 16), jnp.float32),
            pltpu.VMEM((CHUNK, 16), jnp.float32),
            pltpu.VMEM((CHUNK, 16), jnp.float32),
        ],
    )
    return f(dst_blocks)


J_EDGE_C = 45
GRP_E = 9


def _make_mp_async_body(width, jt, grp, edge_split, nbuf, chunk):
    ngrp = jt // grp
    assert jt % grp == 0 and grp % nbuf == 0

    def body(g_hbm, src_hbm, dst_hbm, out_hbm, acc, src_v, dst_v,
             rows, gsems, ssems):
        c = lax.axis_index("c")
        s = lax.axis_index("s")
        tile = c * 16 + s if edge_split else s
        g_c = g_hbm if edge_split else g_hbm.at[c]
        base = s * ROWS_PER_TILE
        sl_all = pl.ds(base, ROWS_PER_TILE)

        if edge_split:
            @pl.when(c == 0)
            def _():
                pltpu.sync_copy(g_c.at[sl_all], acc.at[sl_all])

            @pl.when(c == 1)
            def _():
                _fill_rows(rows[0], chunk, width, 0.0)
                nfull = ROWS_PER_TILE // chunk
                rem = ROWS_PER_TILE - nfull * chunk
                for k in range(nfull):
                    pltpu.sync_copy(
                        rows[0], acc.at[pl.ds(base + k * chunk, chunk)])
                if rem:
                    pltpu.sync_copy(
                        rows[0].at[pl.ds(0, rem)],
                        acc.at[pl.ds(base + nfull * chunk, rem)])
        else:
            pltpu.sync_copy(g_c.at[sl_all], acc.at[sl_all])
        plsc.subcore_barrier()

        def load_grp(grp_i, slot):
            gsl = pl.ds(grp_i * grp, grp)
            osl = pl.ds(slot * grp, grp)
            pltpu.sync_copy(src_hbm.at[tile].at[gsl], src_v.at[osl])
            pltpu.sync_copy(dst_hbm.at[tile].at[gsl], dst_v.at[osl])

        def start_gather(row, b):
            pltpu.async_copy(g_c.at[src_v.at[row]], rows[b], gsems[b])

        def wait_gather(row, b):
            pltpu.make_async_copy(g_c.at[src_v.at[row]], rows[b],
                                  gsems[b]).wait()

        def start_scatter(row, b):
            pltpu.async_copy(rows[b], acc.at[dst_v.at[row]], ssems[b],
                             add=True)

        def wait_scatter(row, b):
            pltpu.make_async_copy(rows[b], acc.at[dst_v.at[row]],
                                  ssems[b]).wait()

        load_grp(0, 0)
        start_gather(0, 0)
        start_gather(1, 1)

        def grp_body(g, _):
            slot = g % 2
            nslot = (g + 1) % 2

            @pl.when(g >= 1)
            def _():
                for d in range(nbuf - 2):
                    wait_scatter(slot * grp, (grp - (nbuf - 2) + d) % nbuf)

            @pl.when(g <= ngrp - 2)
            def _():
                load_grp(g + 1, nslot)

            for j in range(grp):
                b = j % nbuf
                row = slot * grp + j
                wait_gather(row, b)
                start_scatter(row, b)
                nb = (j + 2) % nbuf
                if j + 2 < grp:
                    if j >= nbuf - 2:
                        wait_scatter(row, nb)
                    start_gather(slot * grp + j + 2, nb)
                else:
                    @pl.when(g <= ngrp - 2)
                    def _(j=j, nb=nb, row=row):
                        wait_scatter(row, nb)
                        start_gather(nslot * grp + (j + 2 - grp), nb)
            return 0

        lax.fori_loop(0, ngrp, grp_body, 0)
        last = ((ngrp - 1) % 2) * grp
        for j in range(grp - nbuf, grp):
            wait_scatter(last + j, j % nbuf)

        plsc.subcore_barrier()
        pltpu.sync_copy(acc.at[sl_all], out_hbm.at[c].at[sl_all])

    return body


def _mp_call(width, jt, grp, edge_split, nbuf, chunk):
    return pl.kernel(
        _make_mp_async_body(width, jt, grp, edge_split, nbuf, chunk),
        out_type=jax.ShapeDtypeStruct((2, N_PAD, width), jnp.float32),
        mesh=plsc.VectorSubcoreMesh(**_MESH),
        compiler_params=pltpu.CompilerParams(use_tc_tiling_on_sc=False),
        scratch_types=[
            pltpu.VMEM_SHARED((N_PAD, width), jnp.float32),
            pltpu.VMEM((2 * grp, chunk), jnp.int32),
            pltpu.VMEM((2 * grp, chunk), jnp.int32),
            [pltpu.VMEM((chunk, width), jnp.float32) for _ in range(nbuf)],
            [pltpu.SemaphoreType.DMA for _ in range(nbuf)],
            [pltpu.SemaphoreType.DMA for _ in range(nbuf)],
        ],
    )


def _mp_edge(g, src_blocks, dst_blocks):
    return _mp_call(16, J_EDGE_C, GRP_E, True, 3, CHUNK_C)(
        g, src_blocks, dst_blocks)


def _mp_col(g, src_blocks, dst_blocks):
    return _mp_call(128, J_COL, GRP_COL, False, 3, CHUNK_COL)(
        g, src_blocks, dst_blocks)


def _pre_body(degp_ref, x_ref, dinv_ref, g0_ref):
    p = degp_ref[...]
    deg = 1.0 + p[0, :, 0:1] + p[1, :, 0:1]
    dinv = lax.rsqrt(deg)
    dinv_ref[...] = dinv
    g0_ref[...] = x_ref[...] * dinv


def _pre(degp, x_pad):
    return pl.pallas_call(
        _pre_body,
        grid=(N_BLKS,),
        in_specs=[
            pl.BlockSpec((2, BLK, 16), lambda i: (0, i, 0)),
            pl.BlockSpec((BLK, 16), lambda i: (i, 0)),
        ],
        out_specs=[
            pl.BlockSpec((BLK, 1), lambda i: (i, 0)),
            pl.BlockSpec((BLK, 16), lambda i: (i, 0)),
        ],
        out_shape=[
            jax.ShapeDtypeStruct((N_PAD, 1), jnp.float32),
            jax.ShapeDtypeStruct((N_PAD, 16), jnp.float32),
        ],
    )(degp, x_pad)


def _make_layer_body(first):
    def body(t_ref, dinv_ref, w_ref, b_ref, g_ref):
        p = t_ref[...]
        t = (p[0] + p[1]) if first else jnp.concatenate([p[0], p[1]], axis=1)
        dinv = dinv_ref[...]
        a = t * dinv
        h = jnp.dot(a, w_ref[...], preferred_element_type=jnp.float32,
                    precision=_PREC) + b_ref[...]
        g = jnp.maximum(h, 0.0) * dinv
        g_ref[0, :, :] = g[:, :128]
        g_ref[1, :, :] = g[:, 128:]

    return body


def _layer(t, dinv, w, b, first):
    kin = w.shape[0]
    return pl.pallas_call(
        _make_layer_body(first),
        grid=(N_BLKS,),
        in_specs=[
            pl.BlockSpec((2, BLK, kin if first else 128),
                         lambda i: (0, i, 0)),
            pl.BlockSpec((BLK, 1), lambda i: (i, 0)),
            pl.BlockSpec(w.shape, lambda i: (0, 0)),
            pl.BlockSpec((1, H), lambda i: (0, 0)),
        ],
        out_specs=pl.BlockSpec((2, BLK, 128), lambda i: (0, i, 0)),
        out_shape=jax.ShapeDtypeStruct((2, N_PAD, 128), jnp.float32),
    )(t, dinv, w, b)


def _final_body(t_ref, dinv_ref, w_ref, b_ref, batch_ref, wc1_ref, bc1_ref,
                wc2_ref, bc2_ref, out_ref, sum_s, max_s, cnt_s):
    i = pl.program_id(0)

    @pl.when(i == 0)
    def _():
        sum_s[...] = jnp.zeros_like(sum_s)
        cnt_s[...] = jnp.zeros_like(cnt_s)
        max_s[...] = jnp.full_like(max_s, -jnp.inf)

    p = t_ref[...]
    t = jnp.concatenate([p[0], p[1]], axis=1)
    h = jnp.dot(t * dinv_ref[...], w_ref[...],
                preferred_element_type=jnp.float32, precision=_PREC)
    h = jnp.maximum(h + b_ref[...], 0.0)
    b = batch_ref[...]
    onehot = (b == lax.broadcasted_iota(jnp.int32, (1, G), 1)).astype(
        jnp.float32)
    sum_s[...] += lax.dot_general(onehot, h, (((0,), (0,)), ((), ())),
                                  preferred_element_type=jnp.float32,
                                  precision=_PREC)
    cnt_s[...] += lax.dot_general(onehot, jnp.ones((BLK, 1), jnp.float32),
                                  (((0,), (0,)), ((), ())),
                                  preferred_element_type=jnp.float32,
                                  precision=_PREC)
    def upd_max(g, _):
        col = jnp.max(jnp.where(b == g, h, -jnp.inf), axis=0, keepdims=True)
        max_s[pl.ds(g, 1), :] = jnp.maximum(max_s[pl.ds(g, 1), :], col)
        return 0

    lax.fori_loop(b[0, 0], jnp.minimum(b[BLK - 1, 0], G - 1) + 1, upd_max, 0)

    @pl.when(i == N_BLKS - 1)
    def _():
        mean = sum_s[...] / jnp.maximum(cnt_s[...], 1.0)
        mx = max_s[...]
        mx = jnp.where(jnp.isfinite(mx), mx, 0.0)
        pooled = jnp.concatenate([mean, mx], axis=1)
        hid = jnp.dot(pooled, wc1_ref[...], preferred_element_type=jnp.float32,
                      precision=_PREC) + bc1_ref[...]
        hid = jnp.maximum(hid, 0.0)
        out_ref[...] = jnp.dot(hid, wc2_ref[...],
                               preferred_element_type=jnp.float32,
                               precision=_PREC) + bc2_ref[...]


def _final(t3, dinv, w3, b3, batch_pad, wc1, bc1, wc2p, bc2p):
    return pl.pallas_call(
        _final_body,
        grid=(N_BLKS,),
        in_specs=[
            pl.BlockSpec((2, BLK, 128), lambda i: (0, i, 0)),
            pl.BlockSpec((BLK, 1), lambda i: (i, 0)),
            pl.BlockSpec((H, H), lambda i: (0, 0)),
            pl.BlockSpec((1, H), lambda i: (0, 0)),
            pl.BlockSpec((BLK, 1), lambda i: (i, 0)),
            pl.BlockSpec((2 * H, H), lambda i: (0, 0)),
            pl.BlockSpec((1, H), lambda i: (0, 0)),
            pl.BlockSpec((H, 128), lambda i: (0, 0)),
            pl.BlockSpec((1, 128), lambda i: (0, 0)),
        ],
        out_specs=pl.BlockSpec((G, 128), lambda i: (0, 0)),
        out_shape=jax.ShapeDtypeStruct((G, 128), jnp.float32),
        scratch_shapes=[
            pltpu.VMEM((G, H), jnp.float32),
            pltpu.VMEM((G, H), jnp.float32),
            pltpu.VMEM((G, 1), jnp.float32),
        ],
    )(t3, dinv, w3, b3, batch_pad, wc1, bc1, wc2p, bc2p)


@jax.jit
def kernel(x, edge_index, batch, W1, b1, W2, b2, W3, b3, Wc1, bc1, Wc2, bc2):
    src = edge_index[0]
    dst = edge_index[1]
    pad = jnp.full((E_PAD - E,), N, jnp.int32)
    srcp = jnp.concatenate([src, pad])
    dstp = jnp.concatenate([dst, pad])
    dst_deg = dstp.reshape(32, J_EDGE, CHUNK)
    pad_c = jnp.full((E_PAD_C - E,), N, jnp.int32)
    srcp_c = jnp.concatenate([src, pad_c])
    dstp_c = jnp.concatenate([dst, pad_c])
    src_e = srcp_c.reshape(32, J_EDGE_C, CHUNK_C)
    dst_e = dstp_c.reshape(32, J_EDGE_C, CHUNK_C)
    src_c = srcp_c.reshape(16, J_COL, CHUNK_COL)
    dst_c = dstp_c.reshape(16, J_COL, CHUNK_COL)

    x_pad = jnp.zeros((N_PAD, 16), jnp.float32).at[:N, :F_IN].set(x)
    batch_pad = jnp.full((N_PAD, 1), G, jnp.int32).at[:N, 0].set(batch)
    W1p = jnp.zeros((16, H), jnp.float32).at[:F_IN].set(W1)
    Wc2p = jnp.zeros((H, 128), jnp.float32).at[:, :C].set(Wc2)
    bc2p = jnp.zeros((1, 128), jnp.float32).at[0, :C].set(bc2)

    degp = _deg(dst_deg)
    dinv, g0 = _pre(degp, x_pad)
    t1 = _mp_edge(g0, src_e, dst_e)
    g1 = _layer(t1, dinv, W1p, b1.reshape(1, H), first=True)
    t2 = _mp_col(g1, src_c, dst_c)
    g2 = _layer(t2, dinv, W2, b2.reshape(1, H), first=False)
    t3 = _mp_col(g2, src_c, dst_c)
    out = _final(t3, dinv, W3, b3.reshape(1, H), batch_pad,
                 Wc1, bc1.reshape(1, H), Wc2p, bc2p)
    return out[:, :C]


# speedup vs baseline: 1.5539x; 1.0029x over previous
import functools

import jax
import jax.numpy as jnp
from jax import lax
from jax.experimental import pallas as pl
from jax.experimental.pallas import tpu as pltpu
from jax.experimental.pallas import tpu_sc as plsc

N = 10000
E = 160000
H = 256
G = 16
C = 5
F_IN = 5

N_PAD = 10240
ROWS_PER_TILE = N_PAD // 16
E_PAD = 163840
CHUNK = 128
J_EDGE = 40
CHUNK_C = 112
J_COL = 84
GRP_COL = 6
CHUNK_COL = 120
E_PAD_C = 161280
BLK = 512
N_BLKS = N_PAD // BLK

_MESH = dict(core_axis_name="c", subcore_axis_name="s", num_cores=2,
             num_subcores=16)
_PREC = lax.Precision.DEFAULT


def _fill_rows(ref, nrows, width, value):
    per_row = width // 16

    def body(i, _):
        r = i // per_row
        k = i % per_row
        ref[r, pl.ds(k * 16, 16)] = jnp.full((16,), value, jnp.float32)
        return 0

    lax.fori_loop(0, nrows * per_row, body, 0)


def _deg_body(dst_hbm, degp_hbm, acc, idx_v, ones_v, zbuf, buf):
    c = lax.axis_index("c")
    s = lax.axis_index("s")
    w = c * 16 + s
    _fill_rows(ones_v, CHUNK, 16, 1.0)
    _fill_rows(zbuf, CHUNK, 16, 0.0)
    base = s * ROWS_PER_TILE
    for k in range(ROWS_PER_TILE // CHUNK):
        pltpu.sync_copy(zbuf, acc.at[pl.ds(base + k * CHUNK, CHUNK)])
    plsc.subcore_barrier()
    pltpu.sync_copy(dst_hbm.at[w], idx_v)

    def body(j, _):
        pltpu.sync_copy(ones_v, acc.at[idx_v.at[j]], add=True)
        return 0

    lax.fori_loop(0, J_EDGE, body, 0)
    plsc.subcore_barrier()
    for k in range(ROWS_PER_TILE // CHUNK):
        sl = pl.ds(base + k * CHUNK, CHUNK)
        pltpu.sync_copy(acc.at[sl], buf)
        pltpu.sync_copy(buf, degp_hbm.at[c].at[sl])


def _deg(dst_blocks):
    f = pl.kernel(
        _deg_body,
        out_type=jax.ShapeDtypeStruct((2, N_PAD, 16), jnp.float32),
        mesh=plsc.VectorSubcoreMesh(**_MESH),
        compiler_params=pltpu.CompilerParams(use_tc_tiling_on_sc=False),
        scratch_types=[
            pltpu.VMEM_SHARED((N_PAD, 16), jnp.float32),
            pltpu.VMEM((J_EDGE, CHUNK), jnp.int32),
            pltpu.VMEM((CHUNK, 16), jnp.float32),
            pltpu.VMEM((CHUNK, 16), jnp.float32),
            pltpu.VMEM((CHUNK, 16), jnp.float32),
        ],
    )
    return f(dst_blocks)


J_EDGE_C = 45
GRP_E = 9


def _make_mp_async_body(width, jt, grp, edge_split, nbuf, chunk):
    ngrp = jt // grp
    assert jt % grp == 0 and grp % nbuf == 0

    def body(g_hbm, src_hbm, dst_hbm, out_hbm, acc, src_v, dst_v,
             rows, gsems, ssems):
        c = lax.axis_index("c")
        s = lax.axis_index("s")
        tile = c * 16 + s if edge_split else s
        g_c = g_hbm if edge_split else g_hbm.at[c]
        base = s * ROWS_PER_TILE
        sl_all = pl.ds(base, ROWS_PER_TILE)

        if edge_split:
            @pl.when(c == 0)
            def _():
                pltpu.sync_copy(g_c.at[sl_all], acc.at[sl_all])

            @pl.when(c == 1)
            def _():
                _fill_rows(rows[0], chunk, width, 0.0)
                nfull = ROWS_PER_TILE // chunk
                rem = ROWS_PER_TILE - nfull * chunk
                for k in range(nfull):
                    pltpu.sync_copy(
                        rows[0], acc.at[pl.ds(base + k * chunk, chunk)])
                if rem:
                    pltpu.sync_copy(
                        rows[0].at[pl.ds(0, rem)],
                        acc.at[pl.ds(base + nfull * chunk, rem)])
        else:
            pltpu.sync_copy(g_c.at[sl_all], acc.at[sl_all])
        plsc.subcore_barrier()

        def load_grp(grp_i, slot):
            gsl = pl.ds(grp_i * grp, grp)
            osl = pl.ds(slot * grp, grp)
            pltpu.sync_copy(src_hbm.at[tile].at[gsl], src_v.at[osl])
            pltpu.sync_copy(dst_hbm.at[tile].at[gsl], dst_v.at[osl])

        def start_gather(row, b):
            pltpu.async_copy(g_c.at[src_v.at[row]], rows[b], gsems[b])

        def wait_gather(row, b):
            pltpu.make_async_copy(g_c.at[src_v.at[row]], rows[b],
                                  gsems[b]).wait()

        def start_scatter(row, b):
            pltpu.async_copy(rows[b], acc.at[dst_v.at[row]], ssems[b],
                             add=True)

        def wait_scatter(row, b):
            pltpu.make_async_copy(rows[b], acc.at[dst_v.at[row]],
                                  ssems[b]).wait()

        load_grp(0, 0)
        start_gather(0, 0)
        start_gather(1, 1)

        def grp_body(g, _):
            slot = g % 2
            nslot = (g + 1) % 2

            @pl.when(g >= 1)
            def _():
                for d in range(nbuf - 2):
                    wait_scatter(slot * grp, (grp - (nbuf - 2) + d) % nbuf)

            @pl.when(g <= ngrp - 2)
            def _():
                load_grp(g + 1, nslot)

            for j in range(grp):
                b = j % nbuf
                row = slot * grp + j
                wait_gather(row, b)
                start_scatter(row, b)
                nb = (j + 2) % nbuf
                if j + 2 < grp:
                    if j >= nbuf - 2:
                        wait_scatter(row, nb)
                    start_gather(slot * grp + j + 2, nb)
                else:
                    @pl.when(g <= ngrp - 2)
                    def _(j=j, nb=nb, row=row):
                        wait_scatter(row, nb)
                        start_gather(nslot * grp + (j + 2 - grp), nb)
            return 0

        lax.fori_loop(0, ngrp, grp_body, 0)
        last = ((ngrp - 1) % 2) * grp
        for j in range(grp - nbuf, grp):
            wait_scatter(last + j, j % nbuf)

        plsc.subcore_barrier()
        pltpu.sync_copy(acc.at[sl_all], out_hbm.at[c].at[sl_all])

    return body


def _mp_call(width, jt, grp, edge_split, nbuf, chunk):
    return pl.kernel(
        _make_mp_async_body(width, jt, grp, edge_split, nbuf, chunk),
        out_type=jax.ShapeDtypeStruct((2, N_PAD, width), jnp.float32),
        mesh=plsc.VectorSubcoreMesh(**_MESH),
        compiler_params=pltpu.CompilerParams(use_tc_tiling_on_sc=False),
        scratch_types=[
            pltpu.VMEM_SHARED((N_PAD, width), jnp.float32),
            pltpu.VMEM((2 * grp, chunk), jnp.int32),
            pltpu.VMEM((2 * grp, chunk), jnp.int32),
            [pltpu.VMEM((chunk, width), jnp.float32) for _ in range(nbuf)],
            [pltpu.SemaphoreType.DMA for _ in range(nbuf)],
            [pltpu.SemaphoreType.DMA for _ in range(nbuf)],
        ],
    )


def _mp_edge(g, src_blocks, dst_blocks):
    return _mp_call(16, J_EDGE_C, GRP_E, True, 3, CHUNK_C)(
        g, src_blocks, dst_blocks)


def _mp_col(g, src_blocks, dst_blocks):
    return _mp_call(128, J_COL, GRP_COL, False, 3, CHUNK_COL)(
        g, src_blocks, dst_blocks)


def _pre_body(degp_ref, x_ref, dinv_ref, g0_ref):
    p = degp_ref[...]
    deg = 1.0 + p[0, :, 0:1] + p[1, :, 0:1]
    dinv = lax.rsqrt(deg)
    dinv_ref[...] = dinv
    g0_ref[...] = x_ref[...] * dinv


def _pre(degp, x_pad):
    return pl.pallas_call(
        _pre_body,
        grid=(N_BLKS,),
        in_specs=[
            pl.BlockSpec((2, BLK, 16), lambda i: (0, i, 0)),
            pl.BlockSpec((BLK, 16), lambda i: (i, 0)),
        ],
        out_specs=[
            pl.BlockSpec((BLK, 1), lambda i: (i, 0)),
            pl.BlockSpec((BLK, 16), lambda i: (i, 0)),
        ],
        out_shape=[
            jax.ShapeDtypeStruct((N_PAD, 1), jnp.float32),
            jax.ShapeDtypeStruct((N_PAD, 16), jnp.float32),
        ],
    )(degp, x_pad)


def _make_layer_body(first):
    def body(t_ref, dinv_ref, w_ref, b_ref, g_ref):
        p = t_ref[...]
        t = (p[0] + p[1]) if first else jnp.concatenate([p[0], p[1]], axis=1)
        dinv = dinv_ref[...]
        a = t * dinv
        h = jnp.dot(a, w_ref[...], preferred_element_type=jnp.float32,
                    precision=_PREC) + b_ref[...]
        g = jnp.maximum(h, 0.0) * dinv
        g_ref[0, :, :] = g[:, :128]
        g_ref[1, :, :] = g[:, 128:]

    return body


def _layer(t, dinv, w, b, first):
    kin = w.shape[0]
    return pl.pallas_call(
        _make_layer_body(first),
        grid=(N_BLKS,),
        in_specs=[
            pl.BlockSpec((2, BLK, kin if first else 128),
                         lambda i: (0, i, 0)),
            pl.BlockSpec((BLK, 1), lambda i: (i, 0)),
            pl.BlockSpec(w.shape, lambda i: (0, 0)),
            pl.BlockSpec((1, H), lambda i: (0, 0)),
        ],
        out_specs=pl.BlockSpec((2, BLK, 128), lambda i: (0, i, 0)),
        out_shape=jax.ShapeDtypeStruct((2, N_PAD, 128), jnp.float32),
    )(t, dinv, w, b)


def _final_body(t_ref, dinv_ref, w_ref, b_ref, batch_ref, wc1_ref, bc1_ref,
                wc2_ref, bc2_ref, out_ref, sum_s, max_s, cnt_s):
    i = pl.program_id(0)

    @pl.when(i == 0)
    def _():
        sum_s[...] = jnp.zeros_like(sum_s)
        cnt_s[...] = jnp.zeros_like(cnt_s)
        max_s[...] = jnp.full_like(max_s, -jnp.inf)

    p = t_ref[...]
    t = jnp.concatenate([p[0], p[1]], axis=1)
    h = jnp.dot(t * dinv_ref[...], w_ref[...],
                preferred_element_type=jnp.float32, precision=_PREC)
    h = jnp.maximum(h + b_ref[...], 0.0)
    b = batch_ref[...]
    onehot = (b == lax.broadcasted_iota(jnp.int32, (1, G), 1)).astype(
        jnp.float32)
    sum_s[...] += lax.dot_general(onehot, h, (((0,), (0,)), ((), ())),
                                  preferred_element_type=jnp.float32,
                                  precision=_PREC)
    cnt_s[...] += lax.dot_general(onehot, jnp.ones((BLK, 1), jnp.float32),
                                  (((0,), (0,)), ((), ())),
                                  preferred_element_type=jnp.float32,
                                  precision=_PREC)
    def upd_max(g, _):
        col = jnp.max(jnp.where(b == g, h, -jnp.inf), axis=0, keepdims=True)
        max_s[pl.ds(g, 1), :] = jnp.maximum(max_s[pl.ds(g, 1), :], col)
        return 0

    lax.fori_loop(b[0, 0], jnp.minimum(b[BLK - 1, 0], G - 1) + 1, upd_max, 0)

    @pl.when(i == N_BLKS - 1)
    def _():
        mean = sum_s[...] / jnp.maximum(cnt_s[...], 1.0)
        mx = max_s[...]
        mx = jnp.where(jnp.isfinite(mx), mx, 0.0)
        pooled = jnp.concatenate([mean, mx], axis=1)
        hid = jnp.dot(pooled, wc1_ref[...], preferred_element_type=jnp.float32,
                      precision=_PREC) + bc1_ref[...]
        hid = jnp.maximum(hid, 0.0)
        out_ref[...] = jnp.dot(hid, wc2_ref[...],
                               preferred_element_type=jnp.float32,
                               precision=_PREC) + bc2_ref[...]


def _final(t3, dinv, w3, b3, batch_pad, wc1, bc1, wc2p, bc2p):
    return pl.pallas_call(
        _final_body,
        grid=(N_BLKS,),
        in_specs=[
            pl.BlockSpec((2, BLK, 128), lambda i: (0, i, 0)),
            pl.BlockSpec((BLK, 1), lambda i: (i, 0)),
            pl.BlockSpec((H, H), lambda i: (0, 0)),
            pl.BlockSpec((1, H), lambda i: (0, 0)),
            pl.BlockSpec((BLK, 1), lambda i: (i, 0)),
            pl.BlockSpec((2 * H, H), lambda i: (0, 0)),
            pl.BlockSpec((1, H), lambda i: (0, 0)),
            pl.BlockSpec((H, 128), lambda i: (0, 0)),
            pl.BlockSpec((1, 128), lambda i: (0, 0)),
        ],
        out_specs=pl.BlockSpec((G, 128), lambda i: (0, 0)),
        out_shape=jax.ShapeDtypeStruct((G, 128), jnp.float32),
        scratch_shapes=[
            pltpu.VMEM((G, H), jnp.float32),
            pltpu.VMEM((G, H), jnp.float32),
            pltpu.VMEM((G, 1), jnp.float32),
        ],
    )(t3, dinv, w3, b3, batch_pad, wc1, bc1, wc2p, bc2p)


@jax.jit
def kernel(x, edge_index, batch, W1, b1, W2, b2, W3, b3, Wc1, bc1, Wc2, bc2):
    src = edge_index[0]
    dst = edge_index[1]
    pad = jnp.full((E_PAD - E,), N, jnp.int32)
    srcp = jnp.concatenate([src, pad])
    dstp = jnp.concatenate([dst, pad])
    dst_deg = dstp.reshape(32, J_EDGE, CHUNK)
    pad_c = jnp.full((E_PAD_C - E,), N, jnp.int32)
    srcp_c = jnp.concatenate([src, pad_c])
    dstp_c = jnp.concatenate([dst, pad_c])
    src_e = srcp_c.reshape(32, J_EDGE_C, CHUNK_C)
    dst_e = dstp_c.reshape(32, J_EDGE_C, CHUNK_C)
    src_c = srcp_c.reshape(16, J_COL, CHUNK_COL)
    dst_c = dstp_c.reshape(16, J_COL, CHUNK_COL)

    x_pad = jnp.zeros((N_PAD, 16), jnp.float32).at[:N, :F_IN].set(x)
    batch_pad = jnp.full((N_PAD, 1), G, jnp.int32).at[:N, 0].set(batch)
    W1p = jnp.zeros((16, H), jnp.float32).at[:F_IN].set(W1)
    Wc2p = jnp.zeros((H, 128), jnp.float32).at[:, :C].set(Wc2)
    bc2p = jnp.zeros((1, 128), jnp.float32).at[0, :C].set(bc2)

    degp = _deg(dst_deg)
    dinv, g0 = _pre(degp, x_pad)
    t1 = _mp_edge(g0, src_e, dst_e)
    g1 = _layer(t1, dinv, W1p, b1.reshape(1, H), first=True)
    t2 = _mp_col(g1, src_c, dst_c)
    g2 = _layer(t2, dinv, W2, b2.reshape(1, H), first=False)
    t3 = _mp_col(g2, src_c, dst_c)
    out = _final(t3, dinv, W3, b3.reshape(1, H), batch_pad,
                 Wc1, bc1.reshape(1, H), Wc2p, bc2p)
    return out[:, :C]
